# baseline (speedup 1.0000x reference)
import functools

import jax
import jax.numpy as jnp
from jax.experimental import pallas as pl


def _mlp3_body(inp_ref, w1_ref, b1_ref, w2_ref, b2_ref, w3_ref, b3_ref, out_ref):
    h = inp_ref[:] @ w1_ref[:] + b1_ref[:]
    h = jnp.maximum(h, 0.0)
    h = h @ w2_ref[:] + b2_ref[:]
    h = jnp.maximum(h, 0.0)
    out_ref[:] = h @ w3_ref[:] + b3_ref[:]


@functools.partial(jax.jit, static_argnames=("bm",))
def _mlp3(inp, w1, b1, w2, b2, w3, b3, bm):
    m, din = inp.shape
    hc = w1.shape[1]
    dout = w3.shape[1]
    assert m % bm == 0, (m, bm)
    grid = (m // bm,)
    return pl.pallas_call(
        _mlp3_body,
        grid=grid,
        in_specs=[
            pl.BlockSpec((bm, din), lambda i: (i, 0)),
            pl.BlockSpec((din, hc), lambda i: (0, 0)),
            pl.BlockSpec((1, hc), lambda i: (0, 0)),
            pl.BlockSpec((hc, hc), lambda i: (0, 0)),
            pl.BlockSpec((1, hc), lambda i: (0, 0)),
            pl.BlockSpec((hc, dout), lambda i: (0, 0)),
            pl.BlockSpec((1, dout), lambda i: (0, 0)),
        ],
        out_specs=pl.BlockSpec((bm, dout), lambda i: (i, 0)),
        out_shape=jax.ShapeDtypeStruct((m, dout), jnp.float32),
    )(inp, w1, b1.reshape(1, -1), w2, b2.reshape(1, -1), w3, b3.reshape(1, -1))


def _aggregate(x, dest, ea):
    n = x.shape[0]
    e = ea.shape[0]
    cnt = jax.ops.segment_sum(jnp.ones((e,), jnp.float32), dest, num_segments=n)
    has = (cnt > 0).astype(ea.dtype)
    v = ea[:, 0]
    mx0 = jax.ops.segment_max(v, dest, num_segments=n)
    mn0 = -jax.ops.segment_max(-v, dest, num_segments=n)
    eidx = jnp.arange(e, dtype=jnp.int32)
    max_idx = jax.ops.segment_max(jnp.where(v == mx0[dest], eidx, -1), dest, num_segments=n)
    min_idx = jax.ops.segment_max(jnp.where(v == mn0[dest], eidx, -1), dest, num_segments=n)
    max_ea = ea[jnp.maximum(max_idx, 0)] * has[:, None]
    min_ea = ea[jnp.maximum(min_idx, 0)] * has[:, None]
    hb = cnt > 0
    mx = jnp.where(hb[:, None], jax.ops.segment_max(ea, dest, num_segments=n), 0.0)
    mn = jnp.where(hb[:, None], -jax.ops.segment_max(-ea, dest, num_segments=n), 0.0)
    mean = jax.ops.segment_sum(ea, dest, num_segments=n) / jnp.maximum(cnt, 1.0)[:, None]
    return jnp.concatenate([x, min_ea, max_ea, mn, mean, mx], axis=1)


def kernel(x, edge_index, edge_attr,
           e1_W1, e1_b1, e1_W2, e1_b2, e1_W3, e1_b3,
           n1_W1, n1_b1, n1_W2, n1_b2, n1_W3, n1_b3,
           e2_W1, e2_b1, e2_W2, e2_b2, e2_W3, e2_b3,
           n2_W1, n2_b1, n2_W2, n2_b2, n2_W3, n2_b3):
    row, col = edge_index[0], edge_index[1]

    xs, xd = x[row], x[col]
    inp1 = jnp.concatenate([xs, xd, edge_attr], axis=1)
    m1 = _mlp3(inp1, e1_W1, e1_b1, e1_W2, e1_b2, e1_W3, e1_b3, bm=1280)
    ea1 = jnp.concatenate([xs, xd, edge_attr, m1], axis=1)
    ninp1 = _aggregate(x, col, ea1)
    x1 = _mlp3(ninp1, n1_W1, n1_b1, n1_W2, n1_b2, n1_W3, n1_b3, bm=1000)

    x1s, x1d = x1[row], x1[col]
    inp2 = jnp.concatenate([x1s, x1d, ea1], axis=1)
    m2 = _mlp3(inp2, e2_W1, e2_b1, e2_W2, e2_b2, e2_W3, e2_b3, bm=1280)
    ea2 = jnp.concatenate([x1s, x1d, ea1, m2], axis=1)
    ninp2 = _aggregate(x1, col, ea2)
    x2 = _mlp3(ninp2, n2_W1, n2_b1, n2_W2, n2_b2, n2_W3, n2_b3, bm=1000)
    return x2
